# baseline (speedup 1.0000x reference)
import jax
import jax.numpy as jnp
from jax.experimental import pallas as pl
from jax.experimental.pallas import tpu as pltpu


def _mlp_body(x_ref, eps_ref, w1x_ref, w1e_ref, b1_ref, w2_ref, b2_ref, o_ref,
              w1x_bf, w1e_bf, w2_bf):
    @pl.when(pl.program_id(0) == 0)
    def _convert_weights():
        w1x_bf[...] = w1x_ref[...].astype(jnp.bfloat16)
        w1e_bf[...] = w1e_ref[...].astype(jnp.bfloat16)
        w2_bf[...] = w2_ref[...].astype(jnp.bfloat16)

    h = jnp.dot(x_ref[...].astype(jnp.bfloat16), w1x_bf[...],
                preferred_element_type=jnp.float32)
    h = h + jnp.dot(eps_ref[...].astype(jnp.bfloat16), w1e_bf[...],
                    preferred_element_type=jnp.float32)
    h = jnp.maximum(h + b1_ref[...], 0.0)
    o = jnp.dot(h.astype(jnp.bfloat16), w2_bf[...],
                preferred_element_type=jnp.float32) + b2_ref[...]
    o_ref[...] = jnp.maximum(o, 0.0) + jnp.log1p(jnp.exp(-jnp.abs(o)))


def kernel(x, eps, W1, b1, W2, b2):
    B, d_in = x.shape
    noise_dim = eps.shape[1]
    H = W1.shape[1]
    d_out = W2.shape[1]

    W1x = W1[:d_in]
    W1e = W1[d_in:]
    b1r = b1.reshape(1, H)
    b2r = b2.reshape(1, d_out)

    bm = 256
    grid = (B // bm,)

    return pl.pallas_call(
        _mlp_body,
        grid=grid,
        in_specs=[
            pl.BlockSpec((bm, d_in), lambda i: (i, 0)),
            pl.BlockSpec((bm, noise_dim), lambda i: (i, 0)),
            pl.BlockSpec((d_in, H), lambda i: (0, 0)),
            pl.BlockSpec((noise_dim, H), lambda i: (0, 0)),
            pl.BlockSpec((1, H), lambda i: (0, 0)),
            pl.BlockSpec((H, d_out), lambda i: (0, 0)),
            pl.BlockSpec((1, d_out), lambda i: (0, 0)),
        ],
        out_specs=pl.BlockSpec((bm, d_out), lambda i: (i, 0)),
        out_shape=jax.ShapeDtypeStruct((B, d_out), jnp.float32),
        scratch_shapes=[
            pltpu.VMEM((d_in, H), jnp.bfloat16),
            pltpu.VMEM((noise_dim, H), jnp.bfloat16),
            pltpu.VMEM((H, d_out), jnp.bfloat16),
        ],
    )(x, eps, W1x, W1e, b1r, W2, b2r)


# speedup vs baseline: 1.2178x; 1.2178x over previous
import jax
import jax.numpy as jnp
from jax.experimental import pallas as pl


def _mlp_body(x_ref, eps_ref, w1_ref, b1_ref, w2_ref, b2_ref, o_ref):
    xe = jnp.concatenate(
        [x_ref[...].astype(jnp.bfloat16), eps_ref[...].astype(jnp.bfloat16)],
        axis=1)
    h = jnp.dot(xe, w1_ref[...], preferred_element_type=jnp.float32)
    h = jnp.maximum(h + b1_ref[...], 0.0).astype(jnp.bfloat16)
    o = jnp.dot(h, w2_ref[...], preferred_element_type=jnp.float32)
    o = o + b2_ref[...]
    o_ref[...] = jnp.maximum(o, 0.0) + jnp.log1p(jnp.exp(-jnp.abs(o)))


def kernel(x, eps, W1, b1, W2, b2):
    B, d_in = x.shape
    noise_dim = eps.shape[1]
    H = W1.shape[1]
    d_out = W2.shape[1]

    w1_bf = W1.astype(jnp.bfloat16)
    w2_bf = W2.astype(jnp.bfloat16)
    b1r = b1.reshape(1, H)
    b2r = b2.reshape(1, d_out)

    bm = 512
    grid = (B // bm,)

    return pl.pallas_call(
        _mlp_body,
        grid=grid,
        in_specs=[
            pl.BlockSpec((bm, d_in), lambda i: (i, 0)),
            pl.BlockSpec((bm, noise_dim), lambda i: (i, 0)),
            pl.BlockSpec((d_in + noise_dim, H), lambda i: (0, 0)),
            pl.BlockSpec((1, H), lambda i: (0, 0)),
            pl.BlockSpec((H, d_out), lambda i: (0, 0)),
            pl.BlockSpec((1, d_out), lambda i: (0, 0)),
        ],
        out_specs=pl.BlockSpec((bm, d_out), lambda i: (i, 0)),
        out_shape=jax.ShapeDtypeStruct((B, d_out), jnp.float32),
    )(x, eps, w1_bf, b1r, w2_bf, b2r)


# speedup vs baseline: 1.5034x; 1.2345x over previous
import jax
import jax.numpy as jnp
from jax.experimental import pallas as pl


def _mlp_body(x_ref, eps_ref, w1_ref, b1_ref, w2_ref, b2_ref, o_ref):
    xe = jnp.concatenate([x_ref[...], eps_ref[...]], axis=1)
    h = jnp.dot(xe, w1_ref[...], preferred_element_type=jnp.float32,
                precision=jax.lax.Precision.DEFAULT)
    h = jnp.maximum(h + b1_ref[...], 0.0)
    o = jnp.dot(h, w2_ref[...], preferred_element_type=jnp.float32,
                precision=jax.lax.Precision.DEFAULT)
    o = o + b2_ref[...]
    o_ref[...] = jnp.maximum(o, 0.0) + jnp.log1p(jnp.exp(-jnp.abs(o)))


def kernel(x, eps, W1, b1, W2, b2):
    B, d_in = x.shape
    noise_dim = eps.shape[1]
    H = W1.shape[1]
    d_out = W2.shape[1]

    b1r = b1.reshape(1, H)
    b2r = b2.reshape(1, d_out)

    bm = 512
    grid = (B // bm,)

    return pl.pallas_call(
        _mlp_body,
        grid=grid,
        in_specs=[
            pl.BlockSpec((bm, d_in), lambda i: (i, 0)),
            pl.BlockSpec((bm, noise_dim), lambda i: (i, 0)),
            pl.BlockSpec((d_in + noise_dim, H), lambda i: (0, 0)),
            pl.BlockSpec((1, H), lambda i: (0, 0)),
            pl.BlockSpec((H, d_out), lambda i: (0, 0)),
            pl.BlockSpec((1, d_out), lambda i: (0, 0)),
        ],
        out_specs=pl.BlockSpec((bm, d_out), lambda i: (i, 0)),
        out_shape=jax.ShapeDtypeStruct((B, d_out), jnp.float32),
    )(x, eps, W1, b1r, W2, b2r)
